# baseline (speedup 1.0000x reference)
import functools

import jax
import jax.numpy as jnp
from jax import lax
from jax.experimental import pallas as pl
from jax.experimental.pallas import tpu as pltpu
from jax.experimental.pallas import tpu_sc as plsc

N = 16 * 1 * 512 * 512
NC, NS, L = 2, 16, 16
NW = NC * NS
NE = N // NW
W = 512
NR = N // W
SC_ROWS = 2560
TC_ROWS = NR - SC_ROWS
TC_RB = 1408
RPW = SC_ROWS // NW
RC = 40
NCHUNK = RPW // RC
C = 16384
U = 8

_mesh = plsc.VectorSubcoreMesh(
    core_axis_name="c", subcore_axis_name="s", num_cores=NC, num_subcores=NS)


def _tree_sum(vs):
    while len(vs) > 1:
        vs = [a + b for a, b in zip(vs[::2], vs[1::2])] + (
            [vs[-1]] if len(vs) % 2 else [])
    return vs[0]


def _loss_z(z, t):
    y = jnp.exp(-t)
    w = y / (2.0 + y)
    w2 = w * w
    h = jnp.float32(2.0 / 5.0)
    h = h * w2 + jnp.float32(2.0 / 3.0)
    h = h * w2 + jnp.float32(2.0)
    return jnp.maximum(z, 0.0) + w * h


def _emit(wid, stage, out_hbm, vecs):
    for q, v in enumerate(vecs):
        stage[q] = v
    pltpu.sync_copy(stage, out_hbm.at[wid])


@functools.partial(
    pl.kernel,
    out_type=jax.ShapeDtypeStruct((NW, 2, L), jnp.float32),
    mesh=_mesh,
    compiler_params=pltpu.CompilerParams(use_tc_tiling_on_sc=True),
    scratch_types=[
        pltpu.VMEM((RC, W), jnp.float32),
        pltpu.VMEM((RC, W), jnp.float32),
        pltpu.VMEM((RC, W), jnp.int32),
        pltpu.VMEM((RC, W), jnp.int32),
        pltpu.VMEM((2, L), jnp.float32),
        pltpu.SemaphoreType.DMA,
        pltpu.SemaphoreType.DMA,
        pltpu.SemaphoreType.DMA,
        pltpu.SemaphoreType.DMA,
    ],
)
def _main_kernel(pred_hbm, gt_hbm, out_hbm, pbufa, pbufb, gbufa, gbufb, stage,
                 sp0, sp1, sg0, sg1):
    cid = lax.axis_index("c")
    sid = lax.axis_index("s")
    wid = sid * NC + cid
    r0 = TC_ROWS + wid * RPW
    pbufs = [pbufa, pbufb]
    gbufs = [gbufa, gbufb]
    psems = [sp0, sp1]
    gsems = [sg0, sg1]

    def start(ci):
        par = ci % 2
        sl = pl.ds(r0 + ci * RC, RC)
        return (pltpu.async_copy(pred_hbm.at[sl, :], pbufs[par], psems[par]),
                pltpu.async_copy(gt_hbm.at[sl, :], gbufs[par], gsems[par]))

    GPR = W // (U * L)

    def make_body(par):
        pb, gb = pbufs[par], gbufs[par]

        def body(j, acc):
            tot, cnt = acc
            r = j // GPR
            c0 = (j % GPR) * (U * L)
            ls, gs = [], []
            for u in range(U):
                sl = pl.ds(c0 + u * L, L)
                p = pb[r, sl]
                g = gb[r, sl].astype(jnp.float32)
                s = g * (-2.0) + 1.0
                ls.append(_loss_z(s * p, jnp.abs(p)))
                gs.append(g)
            return (tot + _tree_sum(ls), cnt + _tree_sum(gs))
        return body

    zero = jnp.zeros((L,), jnp.float32)
    tot, cnt = zero, zero
    pending = start(0)
    for ci in range(NCHUNK):
        if ci + 1 < NCHUNK:
            nxt = start(ci + 1)
        for h in pending:
            h.wait()
        tot, cnt = lax.fori_loop(0, RC * (W // (U * L)), make_body(ci % 2),
                                 (tot, cnt))
        if ci + 1 < NCHUNK:
            pending = nxt

    _emit(wid, stage, out_hbm, [tot, cnt])


def _tc_body(pred_ref, gt_ref, tot_ref, cnt_ref):
    i = pl.program_id(0)

    @pl.when(i == 0)
    def _():
        tot_ref[...] = jnp.zeros_like(tot_ref)
        cnt_ref[...] = jnp.zeros_like(cnt_ref)

    p = pred_ref[...]
    g = gt_ref[...].astype(jnp.float32)
    s = g * (-2.0) + 1.0
    l = _loss_z(s * p, jnp.abs(p))
    tot_ref[...] += l.sum(axis=0, keepdims=True)
    cnt_ref[...] += g.sum(axis=0, keepdims=True)


_tc_kernel = pl.pallas_call(
    _tc_body,
    grid=(TC_ROWS // TC_RB,),
    in_specs=[
        pl.BlockSpec((TC_RB, W), lambda i: (i, 0)),
        pl.BlockSpec((TC_RB, W), lambda i: (i, 0)),
    ],
    out_specs=[
        pl.BlockSpec((1, W), lambda i: (0, 0)),
        pl.BlockSpec((1, W), lambda i: (0, 0)),
    ],
    out_shape=[
        jax.ShapeDtypeStruct((1, W), jnp.float32),
        jax.ShapeDtypeStruct((1, W), jnp.float32),
    ],
    compiler_params=pltpu.CompilerParams(
        dimension_semantics=("arbitrary",)),
)


@functools.partial(
    pl.kernel,
    out_type=jax.ShapeDtypeStruct((NW, 3, L), jnp.float32),
    mesh=_mesh,
    scratch_types=[
        pltpu.VMEM((C,), jnp.float32),
        pltpu.VMEM((C,), jnp.int32),
        pltpu.VMEM((L,), jnp.float32),
        pltpu.VMEM((3, L), jnp.float32),
    ],
)
def _count_kernel(pred_hbm, gt_hbm, thr_hbm, out_hbm, pbuf, gbuf, tbuf, stage):
    cid = lax.axis_index("c")
    sid = lax.axis_index("s")
    wid = sid * NC + cid
    base = wid * NE
    zero = jnp.zeros((L,), jnp.float32)
    pltpu.sync_copy(thr_hbm, tbuf)
    thr = tbuf[...]

    def body(j, acc):
        s, c, ps = acc
        ss, cs, pss = [], [], []
        for u in range(U):
            sl = pl.ds(j * (U * L) + u * L, L)
            p = pbuf[sl]
            g = gbuf[sl].astype(jnp.float32)
            sgn = g * (-2.0) + 1.0
            l = _loss_z(sgn * p, jnp.abs(p))
            negl = l - l * g - g
            m = negl > thr
            ss.append(jnp.where(m, negl, 0.0))
            cs.append(jnp.where(m, 1.0, 0.0))
            pss.append(l * g)
        return (s + _tree_sum(ss), c + _tree_sum(cs), ps + _tree_sum(pss))

    s, c, ps = zero, zero, zero
    for ci in range(NCHUNK):
        off = base + ci * C
        pltpu.sync_copy(pred_hbm.at[pl.ds(off, C)], pbuf)
        pltpu.sync_copy(gt_hbm.at[pl.ds(off, C)], gbuf)
        s, c, ps = lax.fori_loop(0, C // (U * L), body, (s, c, ps))

    _emit(wid, stage, out_hbm, [s, c, ps])


def kernel(pred, gt):
    p2 = pred.reshape(NR, W)
    g2 = gt.astype(jnp.int32).reshape(NR, W)
    partials = _main_kernel(p2, g2)
    tc_tot, tc_cnt = _tc_kernel(p2, g2)
    tot = partials[:, 0, :].sum() + tc_tot.sum()
    pos_cnt = (partials[:, 1, :].sum() + tc_cnt.sum()).astype(jnp.int32)
    neg_cnt = N - pos_cnt
    k = jnp.minimum(neg_cnt, (pos_cnt * 3.0).astype(jnp.int32))

    def common(_):
        return tot

    def fallback(_):
        p = pred.reshape(-1)
        g = gt.astype(jnp.int32).reshape(-1)

        def search(_):
            def sbody(i, lohi):
                lo, hi = lohi
                mid = (lo + hi) // 2
                thr = jnp.full((L,), lax.bitcast_convert_type(mid, jnp.float32))
                c = _count_kernel(p, g, thr)[:, 1, :].sum().astype(jnp.int32)
                lt = c < k
                return (jnp.where(lt, lo, mid + 1), jnp.where(lt, mid, hi))

            lo, _hi = lax.fori_loop(
                0, 31, sbody, (jnp.int32(0), jnp.int32(0x7F800000)))
            tstar = lax.bitcast_convert_type(lo, jnp.float32)
            oc = _count_kernel(p, g, jnp.full((L,), tstar))
            s = oc[:, 0, :].sum()
            c = oc[:, 1, :].sum().astype(jnp.int32)
            pos_sum = oc[:, 2, :].sum()
            topk = s + (k - c).astype(jnp.float32) * tstar
            return pos_sum + topk

        def zerof(_):
            oc = _count_kernel(p, g, jnp.zeros((L,), jnp.float32))
            return oc[:, 2, :].sum()

        return lax.cond(k == 0, zerof, search, 0)

    num = lax.cond(k == neg_cnt, common, fallback, 0)
    return num / ((pos_cnt + k).astype(jnp.float32) + 1e-6)


# speedup vs baseline: 1.0333x; 1.0333x over previous
import functools

import jax
import jax.numpy as jnp
from jax import lax
from jax.experimental import pallas as pl
from jax.experimental.pallas import tpu as pltpu
from jax.experimental.pallas import tpu_sc as plsc

N = 16 * 1 * 512 * 512
NC, NS, L = 2, 16, 16
NW = NC * NS
NE = N // NW
W = 512
NR = N // W
SC_ROWS = 2560
TC_ROWS = NR - SC_ROWS
TC_RB = 512
RPW = SC_ROWS // NW
RC = 16
NCHUNK = RPW // RC
C = 16384
NCHUNK_F = NE // C
U = 8

_mesh = plsc.VectorSubcoreMesh(
    core_axis_name="c", subcore_axis_name="s", num_cores=NC, num_subcores=NS)


def _tree_sum(vs):
    while len(vs) > 1:
        vs = [a + b for a, b in zip(vs[::2], vs[1::2])] + (
            [vs[-1]] if len(vs) % 2 else [])
    return vs[0]


def _loss_z(z, t):
    y = jnp.exp(-t)
    w = y / (2.0 + y)
    w2 = w * w
    h = jnp.float32(2.0 / 5.0)
    h = h * w2 + jnp.float32(2.0 / 3.0)
    h = h * w2 + jnp.float32(2.0)
    return jnp.maximum(z, 0.0) + w * h


def _emit(wid, stage, out_hbm, vecs):
    for q, v in enumerate(vecs):
        stage[q] = v
    pltpu.sync_copy(stage, out_hbm.at[wid])


@functools.partial(
    pl.kernel,
    out_type=jax.ShapeDtypeStruct((NW, 2, L), jnp.float32),
    mesh=_mesh,
    compiler_params=pltpu.CompilerParams(use_tc_tiling_on_sc=True),
    scratch_types=[
        pltpu.VMEM((RC, W), jnp.float32),
        pltpu.VMEM((RC, W), jnp.float32),
        pltpu.VMEM((RC, W), jnp.int32),
        pltpu.VMEM((RC, W), jnp.int32),
        pltpu.VMEM((2, L), jnp.float32),
        pltpu.SemaphoreType.DMA,
        pltpu.SemaphoreType.DMA,
        pltpu.SemaphoreType.DMA,
        pltpu.SemaphoreType.DMA,
    ],
)
def _main_kernel(pred_hbm, gt_hbm, out_hbm, pbufa, pbufb, gbufa, gbufb, stage,
                 sp0, sp1, sg0, sg1):
    cid = lax.axis_index("c")
    sid = lax.axis_index("s")
    wid = sid * NC + cid
    r0 = TC_ROWS + wid * RPW
    pbufs = [pbufa, pbufb]
    gbufs = [gbufa, gbufb]
    psems = [sp0, sp1]
    gsems = [sg0, sg1]

    def start(ci):
        par = ci % 2
        sl = pl.ds(r0 + ci * RC, RC)
        return (pltpu.async_copy(pred_hbm.at[sl, :], pbufs[par], psems[par]),
                pltpu.async_copy(gt_hbm.at[sl, :], gbufs[par], gsems[par]))

    GPR = W // (U * L)

    def make_body(par):
        pb, gb = pbufs[par], gbufs[par]

        def body(j, acc):
            tot, cnt = acc
            r = j // GPR
            c0 = (j % GPR) * (U * L)
            ls, gs = [], []
            for u in range(U):
                sl = pl.ds(c0 + u * L, L)
                p = pb[r, sl]
                g = gb[r, sl].astype(jnp.float32)
                s = g * (-2.0) + 1.0
                ls.append(_loss_z(s * p, jnp.abs(p)))
                gs.append(g)
            return (tot + _tree_sum(ls), cnt + _tree_sum(gs))
        return body

    zero = jnp.zeros((L,), jnp.float32)
    tot, cnt = zero, zero
    pending = start(0)
    for ci in range(NCHUNK):
        if ci + 1 < NCHUNK:
            nxt = start(ci + 1)
        for h in pending:
            h.wait()
        tot, cnt = lax.fori_loop(0, RC * (W // (U * L)), make_body(ci % 2),
                                 (tot, cnt))
        if ci + 1 < NCHUNK:
            pending = nxt

    _emit(wid, stage, out_hbm, [tot, cnt])


def _tc_body(pred_ref, gt_ref, tot_ref, cnt_ref):
    i = pl.program_id(0)

    @pl.when(i == 0)
    def _():
        tot_ref[...] = jnp.zeros_like(tot_ref)
        cnt_ref[...] = jnp.zeros_like(cnt_ref)

    p = pred_ref[...]
    g = gt_ref[...].astype(jnp.float32)
    s = g * (-2.0) + 1.0
    l = _loss_z(s * p, jnp.abs(p))
    tot_ref[...] += l.sum(axis=0, keepdims=True)
    cnt_ref[...] += g.sum(axis=0, keepdims=True)


_tc_kernel = pl.pallas_call(
    _tc_body,
    grid=(TC_ROWS // TC_RB,),
    in_specs=[
        pl.BlockSpec((TC_RB, W), lambda i: (i, 0)),
        pl.BlockSpec((TC_RB, W), lambda i: (i, 0)),
    ],
    out_specs=[
        pl.BlockSpec((1, W), lambda i: (0, 0)),
        pl.BlockSpec((1, W), lambda i: (0, 0)),
    ],
    out_shape=[
        jax.ShapeDtypeStruct((1, W), jnp.float32),
        jax.ShapeDtypeStruct((1, W), jnp.float32),
    ],
    compiler_params=pltpu.CompilerParams(
        dimension_semantics=("arbitrary",)),
)


@functools.partial(
    pl.kernel,
    out_type=jax.ShapeDtypeStruct((NW, 3, L), jnp.float32),
    mesh=_mesh,
    scratch_types=[
        pltpu.VMEM((C,), jnp.float32),
        pltpu.VMEM((C,), jnp.int32),
        pltpu.VMEM((L,), jnp.float32),
        pltpu.VMEM((3, L), jnp.float32),
    ],
)
def _count_kernel(pred_hbm, gt_hbm, thr_hbm, out_hbm, pbuf, gbuf, tbuf, stage):
    cid = lax.axis_index("c")
    sid = lax.axis_index("s")
    wid = sid * NC + cid
    base = wid * NE
    zero = jnp.zeros((L,), jnp.float32)
    pltpu.sync_copy(thr_hbm, tbuf)
    thr = tbuf[...]

    def body(j, acc):
        s, c, ps = acc
        ss, cs, pss = [], [], []
        for u in range(U):
            sl = pl.ds(j * (U * L) + u * L, L)
            p = pbuf[sl]
            g = gbuf[sl].astype(jnp.float32)
            sgn = g * (-2.0) + 1.0
            l = _loss_z(sgn * p, jnp.abs(p))
            negl = l - l * g - g
            m = negl > thr
            ss.append(jnp.where(m, negl, 0.0))
            cs.append(jnp.where(m, 1.0, 0.0))
            pss.append(l * g)
        return (s + _tree_sum(ss), c + _tree_sum(cs), ps + _tree_sum(pss))

    s, c, ps = zero, zero, zero
    for ci in range(NCHUNK_F):
        off = base + ci * C
        pltpu.sync_copy(pred_hbm.at[pl.ds(off, C)], pbuf)
        pltpu.sync_copy(gt_hbm.at[pl.ds(off, C)], gbuf)
        s, c, ps = lax.fori_loop(0, C // (U * L), body, (s, c, ps))

    _emit(wid, stage, out_hbm, [s, c, ps])


def kernel(pred, gt):
    p2 = pred.reshape(NR, W)
    g2 = gt.astype(jnp.int32).reshape(NR, W)
    partials = _main_kernel(p2, g2)
    tc_tot, tc_cnt = _tc_kernel(p2, g2)
    tot = partials[:, 0, :].sum() + tc_tot.sum()
    pos_cnt = (partials[:, 1, :].sum() + tc_cnt.sum()).astype(jnp.int32)
    neg_cnt = N - pos_cnt
    k = jnp.minimum(neg_cnt, (pos_cnt * 3.0).astype(jnp.int32))

    def common(_):
        return tot

    def fallback(_):
        p = pred.reshape(-1)
        g = gt.astype(jnp.int32).reshape(-1)

        def search(_):
            def sbody(i, lohi):
                lo, hi = lohi
                mid = (lo + hi) // 2
                thr = jnp.full((L,), lax.bitcast_convert_type(mid, jnp.float32))
                c = _count_kernel(p, g, thr)[:, 1, :].sum().astype(jnp.int32)
                lt = c < k
                return (jnp.where(lt, lo, mid + 1), jnp.where(lt, mid, hi))

            lo, _hi = lax.fori_loop(
                0, 31, sbody, (jnp.int32(0), jnp.int32(0x7F800000)))
            tstar = lax.bitcast_convert_type(lo, jnp.float32)
            oc = _count_kernel(p, g, jnp.full((L,), tstar))
            s = oc[:, 0, :].sum()
            c = oc[:, 1, :].sum().astype(jnp.int32)
            pos_sum = oc[:, 2, :].sum()
            topk = s + (k - c).astype(jnp.float32) * tstar
            return pos_sum + topk

        def zerof(_):
            oc = _count_kernel(p, g, jnp.zeros((L,), jnp.float32))
            return oc[:, 2, :].sum()

        return lax.cond(k == 0, zerof, search, 0)

    num = lax.cond(k == neg_cnt, common, fallback, 0)
    return num / ((pos_cnt + k).astype(jnp.float32) + 1e-6)
